# baseline (speedup 1.0000x reference)
import functools

import jax
import jax.numpy as jnp
from jax import lax
from jax.experimental import pallas as pl
from jax.experimental.pallas import tpu as pltpu
from jax.experimental.pallas import tpu_sc as plsc

N = 16384
B = 64
L = 256
F0 = 94
E = 262144
HD = 376
FINAL = 128
COND = 96 * 107

CH = 128
NW = 32
EPW = E // NW
BLK = 128
NB = EPW // BLK
RNG = 3456
RSIZES = (RNG, RNG, RNG, RNG, N - 4 * RNG)

F0P = 128
D1 = 256
D2 = 384
D3 = 384
HDP = 384
CONDP = 10368
KB = 1152
RB = 1024

_mesh = plsc.VectorSubcoreMesh(core_axis_name="c", subcore_axis_name="s")


def _make_sc_pass(nchunk, deg_mode=False):
    out_type = [jax.ShapeDtypeStruct((2, N, CH), jnp.float32)
                for _ in range(nchunk)]
    scratch = [
        pltpu.VMEM((NB, BLK), jnp.int32),
        pltpu.VMEM((NB, BLK), jnp.int32),
        pltpu.VMEM((NB, BLK), jnp.int32),
        pltpu.VMEM((NB, BLK), jnp.int32),
        pltpu.VMEM((BLK, CH), jnp.float32),
        pltpu.VMEM((BLK, CH), jnp.float32),
        pltpu.VMEM_SHARED((RNG, CH), jnp.float32),
    ]

    def body(edges, zeros_h, *refs):
        hs = refs[:nchunk]
        outs = refs[nchunk:2 * nchunk]
        srcv, dstv, srcl, dstl, buf, zerov, acc = refs[2 * nchunk:]
        c = lax.axis_index("c")
        s = lax.axis_index("s")
        wid = c * 16 + s
        if not deg_mode:
            pltpu.sync_copy(edges.at[0].at[wid], srcv)
        pltpu.sync_copy(edges.at[1].at[wid], dstv)
        pltpu.sync_copy(zeros_h, zerov)

        for h, sz in enumerate(RSIZES):
            base = h * RNG

            def clampf(t, carry):
                i = t // (BLK // 16)
                j = (t % (BLK // 16)) * 16
                d = dstv[i, pl.ds(j, 16)]
                inr = (d >= base) & (d < base + sz)
                if deg_mode:
                    srcl[i, pl.ds(j, 16)] = jnp.where(inr, 0, 1)
                else:
                    sv = srcv[i, pl.ds(j, 16)]
                    srcl[i, pl.ds(j, 16)] = jnp.where(inr, sv, N)
                dstl[i, pl.ds(j, 16)] = jnp.where(inr, d - base, 0)
                return carry

            lax.fori_loop(0, NB * (BLK // 16), clampf, 0)
            zt = sz // 16
            for k in range(nchunk):
                for j in range(zt // BLK):
                    pltpu.sync_copy(zerov,
                                    acc.at[pl.ds(s * zt + j * BLK, BLK)])
                if zt % BLK:
                    pltpu.sync_copy(
                        zerov.at[pl.ds(0, zt % BLK)],
                        acc.at[pl.ds(s * zt + (zt // BLK) * BLK, zt % BLK)])
                plsc.subcore_barrier()

                def step(i, carry):
                    pltpu.sync_copy(hs[k].at[srcl.at[i]], buf)
                    pltpu.sync_copy(buf, acc.at[dstl.at[i]], add=True)
                    return carry

                lax.fori_loop(0, NB, step, 0)
                plsc.subcore_barrier()
                pltpu.sync_copy(
                    acc.at[pl.ds(s * zt, zt)],
                    outs[k].at[c].at[pl.ds(base + s * zt, zt)])
                plsc.subcore_barrier()

    return functools.partial(
        pl.kernel, out_type=out_type, mesh=_mesh, scratch_types=scratch)(body)


_sc_deg = _make_sc_pass(1, deg_mode=True)
_sc_scatter2 = _make_sc_pass(2)
_sc_scatter3 = _make_sc_pass(3)


GROW = N + RB


def _tc_layer1(degp, xp, w1):
    def body(deg_ref, x_ref, w_ref, dinv_ref, h0_ref, h1_ref):
        r = pl.program_id(0)
        tail = r == GROW // RB - 1
        deg = deg_ref[0, :, 0:1] + deg_ref[1, :, 0:1] + 1.0
        dinv = 1.0 / jnp.sqrt(deg)
        h = jnp.dot(x_ref[...], w_ref[...], preferred_element_type=jnp.float32)
        hs = jnp.where(tail, 0.0, h * dinv)
        dinv_ref[...] = dinv
        h0_ref[...] = hs[:, :CH]
        h1_ref[...] = hs[:, CH:]

    last = N // RB - 1
    return pl.pallas_call(
        body,
        grid=(GROW // RB,),
        in_specs=[
            pl.BlockSpec((2, RB, CH), lambda r: (0, jnp.minimum(r, last), 0)),
            pl.BlockSpec((RB, F0P), lambda r: (jnp.minimum(r, last), 0)),
            pl.BlockSpec((F0P, D1), lambda r: (0, 0)),
        ],
        out_specs=[
            pl.BlockSpec((RB, 1), lambda r: (r, 0)),
            pl.BlockSpec((RB, CH), lambda r: (r, 0)),
            pl.BlockSpec((RB, CH), lambda r: (r, 0)),
        ],
        out_shape=[
            jax.ShapeDtypeStruct((GROW, 1), jnp.float32),
            jax.ShapeDtypeStruct((GROW, CH), jnp.float32),
            jax.ShapeDtypeStruct((GROW, CH), jnp.float32),
        ],
    )(degp, xp, w1)


def _make_tc_mid(cin, cout, din, dout):
    def body(*refs):
        dinv_ref = refs[0]
        p_refs = refs[1:1 + cin]
        hs_refs = refs[1 + cin:1 + 2 * cin]
        b_ref = refs[1 + 2 * cin]
        w_ref = refs[2 + 2 * cin]
        out_refs = refs[3 + 2 * cin:]
        r = pl.program_id(0)
        tail = r == GROW // RB - 1
        dinv = dinv_ref[...]
        t = jnp.concatenate(
            [p_refs[k][0] + p_refs[k][1] + hs_refs[k][...]
             for k in range(cin)], axis=1)
        t = jax.nn.relu(t * dinv + b_ref[...])
        hs = jnp.where(
            tail, 0.0,
            jnp.dot(t, w_ref[...], preferred_element_type=jnp.float32) * dinv)
        for k in range(cout):
            out_refs[k][...] = hs[:, k * CH:(k + 1) * CH]

    last = N // RB - 1
    in_specs = [pl.BlockSpec((RB, 1), lambda r: (jnp.minimum(r, last), 0))]
    in_specs += [pl.BlockSpec((2, RB, CH),
                              lambda r: (0, jnp.minimum(r, last), 0))] * cin
    in_specs += [pl.BlockSpec((RB, CH),
                              lambda r: (jnp.minimum(r, last), 0))] * cin
    in_specs += [pl.BlockSpec((1, din), lambda r: (0, 0)),
                 pl.BlockSpec((din, dout), lambda r: (0, 0))]

    def run(dinv, ps, hss, bias, w):
        return pl.pallas_call(
            body,
            grid=(GROW // RB,),
            in_specs=in_specs,
            out_specs=[pl.BlockSpec((RB, CH), lambda r: (r, 0))] * cout,
            out_shape=[jax.ShapeDtypeStruct((GROW, CH), jnp.float32)] * cout,
        )(dinv, *ps, *hss, bias, w)

    return run


_tc_mid_12 = _make_tc_mid(2, 3, D1, D2)
_tc_mid_23 = _make_tc_mid(3, 3, D2, D3)


def _tc_cond(conp, condw, condb, y2):
    def body(con_ref, w_ref, b_ref, y_ref, out_ref):
        k = pl.program_id(0)
        init = b_ref[...] + y_ref[...]
        prev = jnp.where(k == 0, init, out_ref[...])
        out_ref[...] = prev + jnp.dot(
            con_ref[...], w_ref[...], preferred_element_type=jnp.float32)

    return pl.pallas_call(
        body,
        grid=(CONDP // KB,),
        in_specs=[
            pl.BlockSpec((B, KB), lambda k: (0, k)),
            pl.BlockSpec((KB, HDP), lambda k: (k, 0)),
            pl.BlockSpec((1, HDP), lambda k: (0, 0)),
            pl.BlockSpec((B, 1), lambda k: (0, 0)),
        ],
        out_specs=pl.BlockSpec((B, HDP), lambda k: (0, 0)),
        out_shape=jax.ShapeDtypeStruct((B, HDP), jnp.float32),
    )(conp, condw, condb, y2)


def _tc_heads(dinv, p3, hs3, b3, pp, mw, eps_t, ce2):
    def body(*refs):
        dinv_ref = refs[0]
        p_refs = refs[1:4]
        hs_refs = refs[4:7]
        (b_ref, pp_ref, m1w, m1b, m2w, m2b, v1w, v1b, v2w, v2b,
         eps_ref, ce_ref) = refs[7:19]
        d_out, z_out, x2_out, mask_out, kl_out = refs[19:]
        g = pl.program_id(0)
        dinv_b = dinv_ref[...]
        pm = jnp.concatenate(
            [p_refs[k][0] + p_refs[k][1] + hs_refs[k][...]
             for k in range(3)], axis=1)
        pm = pm * dinv_b + b_ref[...]
        xr = jax.nn.relu(pm)
        d = xr + pp_ref[...]
        d_out[...] = d
        h1 = jax.nn.relu(
            jnp.dot(d, m1w[...], preferred_element_type=jnp.float32)
            + m1b[...])
        mu = jnp.dot(h1, m2w[...], preferred_element_type=jnp.float32) \
            + m2b[...]
        h2 = jax.nn.relu(
            jnp.dot(d, v1w[...], preferred_element_type=jnp.float32)
            + v1b[...])
        lv = jnp.dot(h2, v2w[...], preferred_element_type=jnp.float32) \
            + v2b[...]
        zlv = -jnp.abs(lv)
        ez = jnp.exp(zlv)
        blk_sum = jnp.sum(1.0 + zlv - mu * mu - ez)
        z_out[...] = mu + jnp.exp(0.5 * zlv) * eps_ref[...] + ce_ref[0]
        x2_out[...] = jnp.max(xr, axis=0, keepdims=True).reshape(1, 1, HDP)
        mask_out[...] = (xr[:, 0:1] == -999.0).astype(jnp.int32)
        prev = jnp.where(g == 0, jnp.zeros((1, 1), jnp.float32), kl_out[...])
        tot = prev + blk_sum
        kl_out[...] = jnp.where(g == B - 1, tot * (-0.5 / 64.0), tot)

    in_specs = [pl.BlockSpec((L, 1), lambda g: (g, 0))]
    in_specs += [pl.BlockSpec((2, L, CH), lambda g: (0, g, 0))] * 3
    in_specs += [pl.BlockSpec((L, CH), lambda g: (g, 0))] * 3
    in_specs += [pl.BlockSpec((1, HDP), lambda g: (0, 0))] * 2
    in_specs += [
        pl.BlockSpec((HDP, HDP), lambda g: (0, 0)),
        pl.BlockSpec((1, HDP), lambda g: (0, 0)),
    ] * 4
    in_specs += [pl.BlockSpec((L, HDP), lambda g: (g, 0)),
                 pl.BlockSpec((1, 1, HDP), lambda g: (g, 0, 0))]

    return pl.pallas_call(
        body,
        grid=(B,),
        in_specs=in_specs,
        out_specs=[
            pl.BlockSpec((L, HDP), lambda g: (g, 0)),
            pl.BlockSpec((L, HDP), lambda g: (g, 0)),
            pl.BlockSpec((1, 1, HDP), lambda g: (g, 0, 0)),
            pl.BlockSpec((L, 1), lambda g: (g, 0)),
            pl.BlockSpec((1, 1), lambda g: (0, 0)),
        ],
        out_shape=[
            jax.ShapeDtypeStruct((N, HDP), jnp.float32),
            jax.ShapeDtypeStruct((N, HDP), jnp.float32),
            jax.ShapeDtypeStruct((B, 1, HDP), jnp.float32),
            jax.ShapeDtypeStruct((N, 1), jnp.int32),
            jax.ShapeDtypeStruct((1, 1), jnp.float32),
        ],
    )(dinv, *p3, *hs3, b3, pp, *mw, eps_t, ce2)


def _tc_fhead(x2, f1w, f1b, f2w, f2b):
    def body(x_ref, w1_ref, b1_ref, w2_ref, b2_ref, out_ref):
        h = jax.nn.relu(
            jnp.dot(x_ref[...], w1_ref[...],
                    preferred_element_type=jnp.float32) + b1_ref[...])
        out_ref[...] = jnp.dot(
            h, w2_ref[...], preferred_element_type=jnp.float32) + b2_ref[...]

    return pl.pallas_call(
        body,
        out_shape=jax.ShapeDtypeStruct((B, FINAL), jnp.float32),
    )(x2, f1w, f1b, f2w, f2b)


def _pad2(a, r, c):
    return jnp.pad(a, ((0, r - a.shape[0]), (0, c - a.shape[1])))


def kernel(x, edge_index, batch, num_nodes, y, con, eps, params):
    p = params
    f32 = jnp.float32

    xp = _pad2(x, N, F0P)
    w1 = _pad2(p['W1'], F0P, D1)
    w2 = _pad2(p['W2'], D1, D2)
    w3 = _pad2(p['W3'], D2, D3)
    b1 = _pad2(p['b1'].reshape(1, -1), 1, D1)
    b2 = _pad2(p['b2'].reshape(1, -1), 1, D2)
    b3 = _pad2(p['b3'].reshape(1, -1), 1, D3)
    pp = _pad2(p['pp'].reshape(1, -1), 1, HDP)
    mw = []
    for nm in ('m1', 'm2', 'v1', 'v2'):
        mw.append(_pad2(p[nm + 'W'], HDP, HDP))
        mw.append(_pad2(p[nm + 'b'].reshape(1, -1), 1, HDP))
    condw = _pad2(p['condW'], CONDP, HDP)
    condb = _pad2(p['condb'].reshape(1, -1), 1, HDP)
    conp = _pad2(con, B, CONDP)
    f1w = _pad2(p['f1W'], HDP, 1024)
    f1b = p['f1b'].reshape(1, -1)
    f2w = p['f2W']
    f2b = p['f2b'].reshape(1, -1)
    y2 = y.reshape(B, 1)
    eps_t = _pad2(eps.transpose(1, 0, 2).reshape(N, HD), N, HDP)

    edges_r = edge_index.reshape(2, NW, NB, BLK)
    zeros_b = jnp.zeros((BLK, CH), f32)
    table8 = jnp.zeros((8, CH), f32).at[0].set(1.0)

    degp = _sc_deg(edges_r, zeros_b, table8)[0]
    dinv, h10, h11 = _tc_layer1(degp, xp, w1)
    p1 = _sc_scatter2(edges_r, zeros_b, h10, h11)
    hs2 = _tc_mid_12(dinv, p1, (h10, h11), b1, w2)
    p2 = _sc_scatter3(edges_r, zeros_b, *hs2)
    hs3 = _tc_mid_23(dinv, p2, hs2, b2, w3)
    p3 = _sc_scatter3(edges_r, zeros_b, *hs3)
    ce2 = _tc_cond(conp, condw, condb, y2).reshape(B, 1, HDP)
    d_rows, z_rows, x2, mask_rows, kl_arr = _tc_heads(
        dinv, p3, hs3, b3, pp, mw, eps_t, ce2)
    pmvo = _tc_fhead(x2.reshape(B, HDP), f1w, f1b, f2w, f2b)

    d_seq = d_rows[:, :HD].reshape(B, L, HD).transpose(1, 0, 2)
    z = z_rows[:, :HD].reshape(B, L, HD).transpose(1, 0, 2)
    mask = mask_rows.reshape(B, L).astype(bool)
    kl = kl_arr[0, 0]
    return d_seq, z, mask, pmvo, kl
